# Initial kernel scaffold; baseline (speedup 1.0000x reference)
import jax
import jax.numpy as jnp
from jax import lax
from jax.experimental import pallas as pl
from jax.experimental.pallas import tpu as pltpu
from jax.experimental.pallas import tpu_sc as plsc

M = 1024
N = 100000
K = 999
DELTA = 5.0

NW = 32
ROWS_PER_W = M // NW
NVEC = N // 16

BITS_A = 9
BINS_A = 1 << BITS_A
BITS_B = 8
BINS_B = 1 << BITS_B
SH_A = 32 - BITS_A
SH_B = 32 - BITS_A - BITS_B


def _scalar(x):
    if getattr(x, "ndim", 0) == 1:
        return jnp.max(x, axis=0)
    return x


def _extract(vec, j):
    lane = lax.iota(jnp.int32, 16)
    return jnp.sum(jnp.where(lane == j, vec, jnp.zeros_like(vec)), axis=0)


def _key_bits(x):
    u = plsc.bitcast(x, jnp.uint32)
    m = jnp.where((u >> jnp.uint32(31)) == jnp.uint32(1),
                  jnp.uint32(0xFFFFFFFF), jnp.uint32(0x80000000))
    return u ^ m


def _body(x_hbm, t_hbm, out_hbm, row_v, histA, histB, totB, tgt_v, acc_v, sem):
    wid = lax.axis_index("s") * 2 + lax.axis_index("c")
    lane = lax.iota(jnp.int32, 16)
    lane512 = lane * 512
    lane256 = lane * 256
    ones = jnp.ones((16,), jnp.float32)
    zero16 = jnp.zeros((16,), jnp.float32)
    lane0 = lane == 0

    pltpu.sync_copy(t_hbm.at[pl.ds(wid * ROWS_PER_W, ROWS_PER_W)], tgt_v)

    def row_body(i, total_acc):
        row = wid * ROWS_PER_W + i
        pltpu.sync_copy(x_hbm.at[row], row_v)

        def z_a(j, _):
            histA[pl.ds(j * 16, 16)] = zero16
            return 0
        lax.fori_loop(0, (16 * BINS_A) // 16, z_a, 0)

        def z_b(j, _):
            histB[pl.ds(j * 16, 16)] = zero16
            return 0
        lax.fori_loop(0, (3 * 16 * BINS_B) // 16, z_b, 0)

        t = tgt_v[i]
        v16 = plsc.load_gather(row_v, [jnp.zeros((16,), jnp.int32) + t])
        keyv = _key_bits(v16)
        binAv = plsc.bitcast(keyv >> jnp.uint32(SH_A), jnp.int32)
        binBv = plsc.bitcast((keyv >> jnp.uint32(SH_B)) & jnp.uint32(BINS_B - 1),
                             jnp.int32)

        def scan_a(j, _):
            x = row_v[pl.ds(j * 16, 16)]
            key = _key_bits(x)
            binA = plsc.bitcast(key >> jnp.uint32(SH_A), jnp.int32)
            plsc.addupdate_scatter(histA, [binA + lane512], ones)
            return 0
        lax.fori_loop(0, NVEC, scan_a, 0)

        plsc.addupdate_scatter(histA, [binAv + lane512], -ones, mask=lane0)

        def sel_a(gi, carry):
            cum, done, b1, c_above = carry
            g = (BINS_A // 16 - 1) - gi

            def acc_lane(l, a):
                return a + histA[pl.ds(l * 512 + g * 16, 16)]
            w = lax.fori_loop(0, 16, acc_lane, zero16)
            grp = jnp.sum(w, axis=0)
            rev_w = lax.rev(w, (0,))
            s = plsc.cumsum(rev_w)
            cond = (cum + s) >= jnp.float32(K)
            jstar = _scalar(plsc.all_reduce_ffs(cond))
            found_here = jnp.logical_and(jnp.logical_not(done),
                                         (cum + grp) >= jnp.float32(K))
            s_j = _extract(s, jstar)
            w_j = _extract(rev_w, jstar)
            nb1 = jnp.where(found_here, g * 16 + (15 - jstar), b1)
            nca = jnp.where(found_here, cum + s_j - w_j, c_above)
            ncum = jnp.where(done, cum, cum + grp)
            return (ncum, jnp.logical_or(done, found_here), nb1, nca)

        init_a = (jnp.float32(0.0), jnp.bool_(False), jnp.int32(0),
                  jnp.float32(0.0))
        _, _, b1, c_above = lax.fori_loop(0, BINS_A // 16, sel_a, init_a)
        k_rem = jnp.float32(K) - c_above

        b1v = jnp.zeros((16,), jnp.int32) + b1

        def scan_b(j, carry):
            s1, s2 = carry
            x = row_v[pl.ds(j * 16, 16)]
            key = _key_bits(x)
            binA = plsc.bitcast(key >> jnp.uint32(SH_A), jnp.int32)
            above = binA > b1v
            xx = x * x
            s1 = s1 + jnp.where(above, x, zero16)
            s2 = s2 + jnp.where(above, xx, zero16)
            eq = binA == b1v
            binB = plsc.bitcast((key >> jnp.uint32(SH_B)) & jnp.uint32(BINS_B - 1),
                                jnp.int32)
            idx = binB + lane256
            plsc.addupdate_scatter(histB, [idx], ones, mask=eq)
            plsc.addupdate_scatter(histB, [idx + 4096], x, mask=eq)
            plsc.addupdate_scatter(histB, [idx + 8192], xx, mask=eq)
            return (s1, s2)

        S1a_v, S2a_v = lax.fori_loop(0, NVEC, scan_b, (zero16, zero16))

        v_in_b1 = binAv == b1v
        v_above = binAv > b1v
        S1a_v = S1a_v - jnp.where(v_above, v16 * (1.0 / 16.0), zero16)
        S2a_v = S2a_v - jnp.where(v_above, v16 * v16 * (1.0 / 16.0), zero16)
        mb = jnp.logical_and(v_in_b1, lane0)
        idxv = binBv + lane256
        plsc.addupdate_scatter(histB, [idxv], -ones, mask=mb)
        plsc.addupdate_scatter(histB, [idxv + 4096], -v16, mask=mb)
        plsc.addupdate_scatter(histB, [idxv + 8192], -v16 * v16, mask=mb)

        S1a = jnp.sum(S1a_v, axis=0)
        S2a = jnp.sum(S2a_v, axis=0)

        def red_b(gi, _):
            f = gi // (BINS_B // 16)
            g = gi - f * (BINS_B // 16)

            def acc_lane(l, a):
                return a + histB[pl.ds(f * 4096 + l * 256 + g * 16, 16)]
            w = lax.fori_loop(0, 16, acc_lane, zero16)
            totB[pl.ds(f * 256 + g * 16, 16)] = w
            return 0
        lax.fori_loop(0, 3 * (BINS_B // 16), red_b, 0)

        def sel_b(gi, carry):
            cum, done, s1b, s2b = carry
            g = (BINS_B // 16 - 1) - gi
            wc = totB[pl.ds(g * 16, 16)]
            ws = totB[pl.ds(256 + g * 16, 16)]
            wq = totB[pl.ds(512 + g * 16, 16)]
            grp = jnp.sum(wc, axis=0)
            rev_c = lax.rev(wc, (0,))
            rev_s = lax.rev(ws, (0,))
            rev_q = lax.rev(wq, (0,))
            sc = plsc.cumsum(rev_c)
            ss = plsc.cumsum(rev_s)
            sq = plsc.cumsum(rev_q)
            cond = (cum + sc) >= k_rem
            jstar = _scalar(plsc.all_reduce_ffs(cond))
            found_here = jnp.logical_and(jnp.logical_not(done),
                                         (cum + grp) >= k_rem)
            c_j = _extract(sc, jstar)
            w_j = _extract(rev_c, jstar)
            s_j = _extract(ss, jstar)
            q_j = _extract(sq, jstar)
            ws_j = _extract(rev_s, jstar)
            wq_j = _extract(rev_q, jstar)
            r = k_rem - (cum + c_j - w_j)
            mean_b2 = ws_j / w_j
            msq_b2 = wq_j / w_j
            add1 = (s_j - ws_j) + r * mean_b2
            add2 = (q_j - wq_j) + r * msq_b2
            ns1 = jnp.where(found_here, s1b + add1,
                            jnp.where(done, s1b, s1b + jnp.sum(ws, axis=0)))
            ns2 = jnp.where(found_here, s2b + add2,
                            jnp.where(done, s2b, s2b + jnp.sum(wq, axis=0)))
            ncum = jnp.where(done, cum, cum + grp)
            return (ncum, jnp.logical_or(done, found_here), ns1, ns2)

        init_b = (jnp.float32(0.0), jnp.bool_(False),
                  jnp.float32(0.0), jnp.float32(0.0))
        _, _, S1b, S2b = lax.fori_loop(0, BINS_B // 16, sel_b, init_b)

        S1 = S1a + S1b
        S2 = S2a + S2b
        neg_term = (S2 + 2.0 * S1 + jnp.float32(K)) / jnp.float32(K)
        posv = _scalar(v16)
        pos_term = jnp.float32(DELTA) * (posv - 1.0) * (posv - 1.0)
        row_loss = pos_term + neg_term
        return total_acc + jnp.where(lane0, row_loss, 0.0)

    total = lax.fori_loop(0, ROWS_PER_W, row_body, jnp.zeros((16,), jnp.float32))
    acc_v[...] = total
    pltpu.sync_copy(acc_v, out_hbm.at[wid])


def kernel(inputs, targets):
    mesh = plsc.VectorSubcoreMesh(core_axis_name="c", subcore_axis_name="s")
    k = pl.kernel(
        _body,
        out_type=jax.ShapeDtypeStruct((NW, 16), jnp.float32),
        mesh=mesh,
        scratch_types=[
            pltpu.VMEM((N,), jnp.float32),
            pltpu.VMEM((16 * BINS_A,), jnp.float32),
            pltpu.VMEM((3 * 16 * BINS_B,), jnp.float32),
            pltpu.VMEM((3 * BINS_B,), jnp.float32),
            pltpu.VMEM((ROWS_PER_W,), jnp.int32),
            pltpu.VMEM((16,), jnp.float32),
            pltpu.SemaphoreType.DMA,
        ],
    )
    parts = k(inputs, targets.astype(jnp.int32))
    return jnp.sum(parts) / jnp.float32(M)


# speedup vs baseline: 11.4327x; 11.4327x over previous
import jax
import jax.numpy as jnp
from jax import lax
from jax.experimental import pallas as pl
from jax.experimental.pallas import tpu as pltpu
from jax.experimental.pallas import tpu_sc as plsc

M = 1024
N = 100000
K = 999
DELTA = 5.0

NW = 32
ROWS_PER_W = M // NW
NVEC = N // 16

BITS_A = 9
BINS_A = 1 << BITS_A
BITS_B = 8
BINS_B = 1 << BITS_B
SH_A = 32 - BITS_A
SH_B = 32 - BITS_A - BITS_B


def _scalar(x):
    if getattr(x, "ndim", 0) == 1:
        return jnp.max(x, axis=0)
    return x


def _extract(vec, j):
    lane = lax.iota(jnp.int32, 16)
    return jnp.sum(jnp.where(lane == j, vec, jnp.zeros_like(vec)), axis=0)


def _div(a, b):
    z = jnp.zeros((16,), jnp.float32)
    return jnp.max((z + a) / (z + b), axis=0)


def _key_bits(x):
    u = plsc.bitcast(x, jnp.uint32)
    m = jnp.where((u >> jnp.uint32(31)) == jnp.uint32(1),
                  jnp.uint32(0xFFFFFFFF), jnp.uint32(0x80000000))
    return u ^ m


def _body(x_hbm, t_hbm, out_hbm, row_v, histA, histB, totB, tgt_v, acc_v, sem):
    wid = lax.axis_index("s") * 2 + lax.axis_index("c")
    lane = lax.iota(jnp.int32, 16)
    lane512 = lane * 512
    lane256 = lane * 256
    ones = jnp.ones((16,), jnp.float32)
    zero16 = jnp.zeros((16,), jnp.float32)
    lane0 = lane == 0

    pltpu.sync_copy(t_hbm.at[pl.ds(wid * ROWS_PER_W, ROWS_PER_W)], tgt_v)

    def row_body(i, total_acc):
        row = wid * ROWS_PER_W + i
        pltpu.sync_copy(x_hbm.at[row], row_v)

        def z_a(j, _):
            histA[pl.ds(j * 16, 16)] = zero16
            return 0
        lax.fori_loop(0, (16 * BINS_A) // 16, z_a, 0)

        def z_b(j, _):
            histB[pl.ds(j * 16, 16)] = zero16
            return 0
        lax.fori_loop(0, (3 * 16 * BINS_B) // 16, z_b, 0)

        g16 = (i // 16) * 16
        tvec = tgt_v[pl.ds(g16, 16)].astype(jnp.float32)
        t = jnp.sum(jnp.where(lane == (i - g16), tvec, jnp.zeros_like(tvec)),
                    axis=0).astype(jnp.int32)
        v16 = plsc.load_gather(row_v, [jnp.zeros((16,), jnp.int32) + t])
        keyv = _key_bits(v16)
        binAv = plsc.bitcast(keyv >> jnp.uint32(SH_A), jnp.int32)
        binBv = plsc.bitcast((keyv >> jnp.uint32(SH_B)) & jnp.uint32(BINS_B - 1),
                             jnp.int32)

        def scan_a(j, _):
            x = row_v[pl.ds(j * 16, 16)]
            key = _key_bits(x)
            binA = plsc.bitcast(key >> jnp.uint32(SH_A), jnp.int32)
            plsc.addupdate_scatter(histA, [binA + lane512], ones)
            return 0
        lax.fori_loop(0, NVEC, scan_a, 0)

        plsc.addupdate_scatter(histA, [binAv + lane512], -ones, mask=lane0)

        def sel_a(gi, carry):
            cum, done, b1, c_above = carry
            g = (BINS_A // 16 - 1) - gi

            def acc_lane(l, a):
                return a + histA[pl.ds(l * 512 + g * 16, 16)]
            w = lax.fori_loop(0, 16, acc_lane, zero16)
            grp = jnp.sum(w, axis=0)
            rev_w = lax.rev(w, (0,))
            s = plsc.cumsum(rev_w)
            cond = (cum + s) >= jnp.float32(K)
            jstar = _scalar(plsc.all_reduce_ffs(cond))
            found_here = jnp.logical_and(jnp.logical_not(done),
                                         (cum + grp) >= jnp.float32(K))
            s_j = _extract(s, jstar)
            w_j = _extract(rev_w, jstar)
            nb1 = jnp.where(found_here, g * 16 + (15 - jstar), b1)
            nca = jnp.where(found_here, cum + s_j - w_j, c_above)
            ncum = jnp.where(done, cum, cum + grp)
            return (ncum, jnp.logical_or(done, found_here), nb1, nca)

        init_a = (jnp.float32(0.0), jnp.bool_(False), jnp.int32(0),
                  jnp.float32(0.0))
        _, _, b1, c_above = lax.fori_loop(0, BINS_A // 16, sel_a, init_a)
        k_rem = jnp.float32(K) - c_above

        b1v = jnp.zeros((16,), jnp.int32) + b1

        def scan_b(j, carry):
            s1, s2 = carry
            x = row_v[pl.ds(j * 16, 16)]
            key = _key_bits(x)
            binA = plsc.bitcast(key >> jnp.uint32(SH_A), jnp.int32)
            above = binA > b1v
            xx = x * x
            s1 = s1 + jnp.where(above, x, zero16)
            s2 = s2 + jnp.where(above, xx, zero16)
            eq = binA == b1v
            binB = plsc.bitcast((key >> jnp.uint32(SH_B)) & jnp.uint32(BINS_B - 1),
                                jnp.int32)
            idx = binB + lane256
            plsc.addupdate_scatter(histB, [idx], ones, mask=eq)
            plsc.addupdate_scatter(histB, [idx + 4096], x, mask=eq)
            plsc.addupdate_scatter(histB, [idx + 8192], xx, mask=eq)
            return (s1, s2)

        S1a_v, S2a_v = lax.fori_loop(0, NVEC, scan_b, (zero16, zero16))

        v_in_b1 = binAv == b1v
        v_above = binAv > b1v
        S1a_v = S1a_v - jnp.where(v_above, v16 * (1.0 / 16.0), zero16)
        S2a_v = S2a_v - jnp.where(v_above, v16 * v16 * (1.0 / 16.0), zero16)
        mb = jnp.logical_and(v_in_b1, lane0)
        idxv = binBv + lane256
        plsc.addupdate_scatter(histB, [idxv], -ones, mask=mb)
        plsc.addupdate_scatter(histB, [idxv + 4096], -v16, mask=mb)
        plsc.addupdate_scatter(histB, [idxv + 8192], -v16 * v16, mask=mb)

        S1a = jnp.sum(S1a_v, axis=0)
        S2a = jnp.sum(S2a_v, axis=0)

        def red_b(gi, _):
            f = gi // (BINS_B // 16)
            g = gi - f * (BINS_B // 16)

            def acc_lane(l, a):
                return a + histB[pl.ds(f * 4096 + l * 256 + g * 16, 16)]
            w = lax.fori_loop(0, 16, acc_lane, zero16)
            totB[pl.ds(f * 256 + g * 16, 16)] = w
            return 0
        lax.fori_loop(0, 3 * (BINS_B // 16), red_b, 0)

        def sel_b(gi, carry):
            cum, done, s1b, s2b = carry
            g = (BINS_B // 16 - 1) - gi
            wc = totB[pl.ds(g * 16, 16)]
            ws = totB[pl.ds(256 + g * 16, 16)]
            wq = totB[pl.ds(512 + g * 16, 16)]
            grp = jnp.sum(wc, axis=0)
            rev_c = lax.rev(wc, (0,))
            rev_s = lax.rev(ws, (0,))
            rev_q = lax.rev(wq, (0,))
            sc = plsc.cumsum(rev_c)
            ss = plsc.cumsum(rev_s)
            sq = plsc.cumsum(rev_q)
            cond = (cum + sc) >= k_rem
            jstar = _scalar(plsc.all_reduce_ffs(cond))
            found_here = jnp.logical_and(jnp.logical_not(done),
                                         (cum + grp) >= k_rem)
            c_j = _extract(sc, jstar)
            w_j = _extract(rev_c, jstar)
            s_j = _extract(ss, jstar)
            q_j = _extract(sq, jstar)
            ws_j = _extract(rev_s, jstar)
            wq_j = _extract(rev_q, jstar)
            r = k_rem - (cum + c_j - w_j)
            mean_b2 = _div(ws_j, w_j)
            msq_b2 = _div(wq_j, w_j)
            add1 = (s_j - ws_j) + r * mean_b2
            add2 = (q_j - wq_j) + r * msq_b2
            ns1 = jnp.where(found_here, s1b + add1,
                            jnp.where(done, s1b, s1b + jnp.sum(ws, axis=0)))
            ns2 = jnp.where(found_here, s2b + add2,
                            jnp.where(done, s2b, s2b + jnp.sum(wq, axis=0)))
            ncum = jnp.where(done, cum, cum + grp)
            return (ncum, jnp.logical_or(done, found_here), ns1, ns2)

        init_b = (jnp.float32(0.0), jnp.bool_(False),
                  jnp.float32(0.0), jnp.float32(0.0))
        _, _, S1b, S2b = lax.fori_loop(0, BINS_B // 16, sel_b, init_b)

        S1 = S1a + S1b
        S2 = S2a + S2b
        neg_term = (S2 + 2.0 * S1 + jnp.float32(K)) * jnp.float32(1.0 / K)
        posv = _scalar(v16)
        pos_term = jnp.float32(DELTA) * (posv - 1.0) * (posv - 1.0)
        row_loss = pos_term + neg_term
        return total_acc + jnp.where(lane0, row_loss, 0.0)

    total = lax.fori_loop(0, ROWS_PER_W, row_body, jnp.zeros((16,), jnp.float32))
    acc_v[...] = total
    pltpu.sync_copy(acc_v, out_hbm.at[wid])


def kernel(inputs, targets):
    mesh = plsc.VectorSubcoreMesh(core_axis_name="c", subcore_axis_name="s")
    k = pl.kernel(
        _body,
        out_type=jax.ShapeDtypeStruct((NW, 16), jnp.float32),
        mesh=mesh,
        scratch_types=[
            pltpu.VMEM((N,), jnp.float32),
            pltpu.VMEM((16 * BINS_A,), jnp.float32),
            pltpu.VMEM((3 * 16 * BINS_B,), jnp.float32),
            pltpu.VMEM((3 * BINS_B,), jnp.float32),
            pltpu.VMEM((ROWS_PER_W,), jnp.int32),
            pltpu.VMEM((16,), jnp.float32),
            pltpu.SemaphoreType.DMA,
        ],
        compiler_params=pltpu.CompilerParams(needs_layout_passes=False),
    )
    parts = k(inputs, targets.astype(jnp.int32))
    return jnp.sum(parts) / jnp.float32(M)


# speedup vs baseline: 11.7684x; 1.0294x over previous
import jax
import jax.numpy as jnp
from jax import lax
from jax.experimental import pallas as pl
from jax.experimental.pallas import tpu as pltpu
from jax.experimental.pallas import tpu_sc as plsc

M = 1024
N = 100000
K = 999
DELTA = 5.0

NW = 32
ROWS_PER_W = M // NW
NVEC = N // 16
UNROLL = 10

BITS_A = 9
BINS_A = 1 << BITS_A
BITS_B = 8
BINS_B = 1 << BITS_B
SH_A = 32 - BITS_A
SH_B = 32 - BITS_A - BITS_B


def _scalar(x):
    if getattr(x, "ndim", 0) == 1:
        return jnp.max(x, axis=0)
    return x


def _extract(vec, j):
    lane = lax.iota(jnp.int32, 16)
    return jnp.sum(jnp.where(lane == j, vec, jnp.zeros_like(vec)), axis=0)


def _key_bits(x):
    u = plsc.bitcast(x, jnp.uint32)
    m = jnp.where((u >> jnp.uint32(31)) == jnp.uint32(1),
                  jnp.uint32(0xFFFFFFFF), jnp.uint32(0x80000000))
    return u ^ m


def _body(x_hbm, t_hbm, out_hbm, row_v, histA, histB, totB, tgt_v, acc_v, sem):
    wid = lax.axis_index("s") * 2 + lax.axis_index("c")
    lane = lax.iota(jnp.int32, 16)
    lane512 = lane * 512
    lane256 = lane * 256
    ones = jnp.ones((16,), jnp.float32)
    zero16 = jnp.zeros((16,), jnp.float32)
    lane0 = lane == 0
    kf = jnp.float32(K)

    pltpu.sync_copy(t_hbm.at[pl.ds(wid * ROWS_PER_W, ROWS_PER_W)], tgt_v)

    def z_a(j, _):
        histA[pl.ds(j * 16, 16)] = zero16
        return 0
    lax.fori_loop(0, (16 * BINS_A) // 16, z_a, 0)

    def z_b(j, _):
        histB[pl.ds(j * 16, 16)] = zero16
        return 0
    lax.fori_loop(0, (2 * 16 * BINS_B) // 16, z_b, 0)

    def row_body(i, total_acc):
        row = wid * ROWS_PER_W + i
        pltpu.sync_copy(x_hbm.at[row], row_v)

        g16 = (i // 16) * 16
        tvec = tgt_v[pl.ds(g16, 16)].astype(jnp.float32)
        t = jnp.sum(jnp.where(lane == (i - g16), tvec, jnp.zeros_like(tvec)),
                    axis=0).astype(jnp.int32)
        v16 = plsc.load_gather(row_v, [jnp.zeros((16,), jnp.int32) + t])
        keyv = _key_bits(v16)
        binAv = plsc.bitcast(keyv >> jnp.uint32(SH_A), jnp.int32)
        binBv = plsc.bitcast((keyv >> jnp.uint32(SH_B)) & jnp.uint32(BINS_B - 1),
                             jnp.int32)

        def scan_a(j, _):
            base = j * (16 * UNROLL)
            for u in range(UNROLL):
                x = row_v[pl.ds(base + u * 16, 16)]
                key = _key_bits(x)
                binA = plsc.bitcast(key >> jnp.uint32(SH_A), jnp.int32)
                plsc.addupdate_scatter(histA, [binA + lane512], ones)
            return 0
        lax.fori_loop(0, NVEC // UNROLL, scan_a, 0)

        plsc.addupdate_scatter(histA, [binAv + lane512], -ones, mask=lane0)

        def sel_a(gi, carry):
            cum, done, b1, c_above = carry
            g = (BINS_A // 16 - 1) - gi
            w = zero16
            for l in range(16):
                sl = pl.ds(l * 512 + g * 16, 16)
                w = w + histA[sl]
                histA[sl] = zero16
            grp = jnp.sum(w, axis=0)
            rev_w = lax.rev(w, (0,))
            s = plsc.cumsum(rev_w)
            cond = (cum + s) >= kf
            jstar = _scalar(plsc.all_reduce_ffs(cond))
            found_here = jnp.logical_and(jnp.logical_not(done),
                                         (cum + grp) >= kf)
            s_j = _extract(s, jstar)
            w_j = _extract(rev_w, jstar)
            nb1 = jnp.where(found_here, g * 16 + (15 - jstar), b1)
            nca = jnp.where(found_here, cum + s_j - w_j, c_above)
            ncum = jnp.where(done, cum, cum + grp)
            return (ncum, jnp.logical_or(done, found_here), nb1, nca)

        init_a = (jnp.float32(0.0), jnp.bool_(False), jnp.int32(0),
                  jnp.float32(0.0))
        _, _, b1, c_above = lax.fori_loop(0, BINS_A // 16, sel_a, init_a)
        k_rem = kf - c_above

        b1v = jnp.zeros((16,), jnp.int32) + b1

        def scan_b(j, carry):
            s1a, s2a, s1b, s2b = carry
            accs = [[s1a, s2a], [s1b, s2b]]
            base = j * (16 * UNROLL)
            for u in range(UNROLL):
                x = row_v[pl.ds(base + u * 16, 16)]
                key = _key_bits(x)
                binA = plsc.bitcast(key >> jnp.uint32(SH_A), jnp.int32)
                above = binA > b1v
                xx = x * x
                a = accs[u % 2]
                a[0] = a[0] + jnp.where(above, x, zero16)
                a[1] = a[1] + jnp.where(above, xx, zero16)
                eq = binA == b1v
                binB = plsc.bitcast(
                    (key >> jnp.uint32(SH_B)) & jnp.uint32(BINS_B - 1),
                    jnp.int32)
                idx = binB + lane256
                plsc.addupdate_scatter(histB, [idx], ones, mask=eq)
                plsc.addupdate_scatter(histB, [idx + 4096], x, mask=eq)
            return (accs[0][0], accs[0][1], accs[1][0], accs[1][1])

        z4 = (zero16, zero16, zero16, zero16)
        r1, r2, r3, r4 = lax.fori_loop(0, NVEC // UNROLL, scan_b, z4)
        S1a_v = r1 + r3
        S2a_v = r2 + r4

        v_in_b1 = binAv == b1v
        v_above = binAv > b1v
        S1a_v = S1a_v - jnp.where(v_above, v16 * (1.0 / 16.0), zero16)
        S2a_v = S2a_v - jnp.where(v_above, v16 * v16 * (1.0 / 16.0), zero16)
        mb = jnp.logical_and(v_in_b1, lane0)
        idxv = binBv + lane256
        plsc.addupdate_scatter(histB, [idxv], -ones, mask=mb)
        plsc.addupdate_scatter(histB, [idxv + 4096], -v16, mask=mb)

        S1a = jnp.sum(S1a_v, axis=0)
        S2a = jnp.sum(S2a_v, axis=0)

        def red_b(g, _):
            for f in range(2):
                w = zero16
                for l in range(16):
                    sl = pl.ds(f * 4096 + l * 256 + g * 16, 16)
                    w = w + histB[sl]
                    histB[sl] = zero16
                totB[pl.ds(f * 256 + g * 16, 16)] = w
            return 0
        lax.fori_loop(0, BINS_B // 16, red_b, 0)

        def sel_b(gi, carry):
            cum, done, s1b, s2b = carry
            g = (BINS_B // 16 - 1) - gi
            wc = totB[pl.ds(g * 16, 16)]
            ws = totB[pl.ds(256 + g * 16, 16)]
            mean = ws / jnp.maximum(wc, ones)
            wq = ws * mean
            grp = jnp.sum(wc, axis=0)
            rev_c = lax.rev(wc, (0,))
            rev_s = lax.rev(ws, (0,))
            rev_q = lax.rev(wq, (0,))
            rev_m = lax.rev(mean, (0,))
            sc = plsc.cumsum(rev_c)
            ss = plsc.cumsum(rev_s)
            sq = plsc.cumsum(rev_q)
            cond = (cum + sc) >= k_rem
            jstar = _scalar(plsc.all_reduce_ffs(cond))
            found_here = jnp.logical_and(jnp.logical_not(done),
                                         (cum + grp) >= k_rem)
            c_j = _extract(sc, jstar)
            w_j = _extract(rev_c, jstar)
            s_j = _extract(ss, jstar)
            q_j = _extract(sq, jstar)
            ws_j = _extract(rev_s, jstar)
            wq_j = _extract(rev_q, jstar)
            m_j = _extract(rev_m, jstar)
            r = k_rem - (cum + c_j - w_j)
            add1 = (s_j - ws_j) + r * m_j
            add2 = (q_j - wq_j) + r * m_j * m_j
            ns1 = jnp.where(found_here, s1b + add1,
                            jnp.where(done, s1b, s1b + jnp.sum(ws, axis=0)))
            ns2 = jnp.where(found_here, s2b + add2,
                            jnp.where(done, s2b, s2b + jnp.sum(wq, axis=0)))
            ncum = jnp.where(done, cum, cum + grp)
            return (ncum, jnp.logical_or(done, found_here), ns1, ns2)

        init_b = (jnp.float32(0.0), jnp.bool_(False),
                  jnp.float32(0.0), jnp.float32(0.0))
        _, _, S1b, S2b = lax.fori_loop(0, BINS_B // 16, sel_b, init_b)

        S1 = S1a + S1b
        S2 = S2a + S2b
        neg_term = (S2 + 2.0 * S1 + kf) * jnp.float32(1.0 / K)
        posv = _scalar(v16)
        pos_term = jnp.float32(DELTA) * (posv - 1.0) * (posv - 1.0)
        row_loss = pos_term + neg_term
        return total_acc + jnp.where(lane0, row_loss, 0.0)

    total = lax.fori_loop(0, ROWS_PER_W, row_body, jnp.zeros((16,), jnp.float32))
    acc_v[...] = total
    pltpu.sync_copy(acc_v, out_hbm.at[wid])


def kernel(inputs, targets):
    mesh = plsc.VectorSubcoreMesh(core_axis_name="c", subcore_axis_name="s")
    k = pl.kernel(
        _body,
        out_type=jax.ShapeDtypeStruct((NW, 16), jnp.float32),
        mesh=mesh,
        scratch_types=[
            pltpu.VMEM((N,), jnp.float32),
            pltpu.VMEM((16 * BINS_A,), jnp.float32),
            pltpu.VMEM((2 * 16 * BINS_B,), jnp.float32),
            pltpu.VMEM((2 * BINS_B,), jnp.float32),
            pltpu.VMEM((ROWS_PER_W,), jnp.int32),
            pltpu.VMEM((16,), jnp.float32),
            pltpu.SemaphoreType.DMA,
        ],
        compiler_params=pltpu.CompilerParams(needs_layout_passes=False),
    )
    parts = k(inputs, targets.astype(jnp.int32))
    return jnp.sum(parts) / jnp.float32(M)


# speedup vs baseline: 31.2247x; 2.6533x over previous
import jax
import jax.numpy as jnp
from jax import lax
from jax.experimental import pallas as pl
from jax.experimental.pallas import tpu as pltpu
from jax.experimental.pallas import tpu_sc as plsc

M = 1024
N = 100000
K = 999
DELTA = 5.0

NW = 32
ROWS_PER_W = M // NW
NVEC = N // 16
UNROLL = 10

BITS_A = 9
BINS_A = 1 << BITS_A
BITS_B = 8
BINS_B = 1 << BITS_B
SH_A = 32 - BITS_A
SH_B = 32 - BITS_A - BITS_B


def _scalar(x):
    if getattr(x, "ndim", 0) == 1:
        return jnp.max(x, axis=0)
    return x


def _extract(vec, j):
    lane = lax.iota(jnp.int32, 16)
    return jnp.sum(jnp.where(lane == j, vec, jnp.zeros_like(vec)), axis=0)


def _key_bits(x):
    u = plsc.bitcast(x, jnp.uint32)
    m = jnp.where((u >> jnp.uint32(31)) == jnp.uint32(1),
                  jnp.uint32(0xFFFFFFFF), jnp.uint32(0x80000000))
    return u ^ m


def _body(x_hbm, t_hbm, out_hbm, row_v, histA, histB, totB, tgt_v, acc_v, sem):
    wid = lax.axis_index("s") * 2 + lax.axis_index("c")
    lane = lax.iota(jnp.int32, 16)
    lane512 = lane * 512
    lane256 = lane * 256
    ones = jnp.ones((16,), jnp.float32)
    zero16 = jnp.zeros((16,), jnp.float32)
    lane0 = lane == 0
    kf = jnp.float32(K)

    pltpu.sync_copy(t_hbm.at[pl.ds(wid * ROWS_PER_W, ROWS_PER_W)], tgt_v)

    @plsc.parallel_loop(0, (16 * BINS_A) // 16, unroll=8)
    def z_a(j):
        histA[pl.ds(j * 16, 16)] = zero16

    @plsc.parallel_loop(0, (2 * 16 * BINS_B) // 16, unroll=8)
    def z_b(j):
        histB[pl.ds(j * 16, 16)] = zero16

    def row_body(i, total_acc):
        row = wid * ROWS_PER_W + i
        pltpu.sync_copy(x_hbm.at[row], row_v)

        g16 = (i // 16) * 16
        tvec = tgt_v[pl.ds(g16, 16)].astype(jnp.float32)
        t = jnp.sum(jnp.where(lane == (i - g16), tvec, jnp.zeros_like(tvec)),
                    axis=0).astype(jnp.int32)
        v16 = plsc.load_gather(row_v, [jnp.zeros((16,), jnp.int32) + t])
        keyv = _key_bits(v16)
        binAv = plsc.bitcast(keyv >> jnp.uint32(SH_A), jnp.int32)
        binBv = plsc.bitcast((keyv >> jnp.uint32(SH_B)) & jnp.uint32(BINS_B - 1),
                             jnp.int32)

        @plsc.parallel_loop(0, NVEC, step=UNROLL)
        def scan_a(j):
            for u in range(UNROLL):
                x = row_v[pl.ds(j * 16 + u * 16, 16)]
                key = _key_bits(x)
                binA = plsc.bitcast(key >> jnp.uint32(SH_A), jnp.int32)
                plsc.addupdate_scatter(histA, [binA + lane512], ones)

        plsc.addupdate_scatter(histA, [binAv + lane512], -ones, mask=lane0)

        init_a = (jnp.float32(0.0), jnp.bool_(False), jnp.int32(0),
                  jnp.float32(0.0))

        @plsc.parallel_loop(0, BINS_A // 16, carry=init_a)
        def sel_a(gi, carry):
            cum, done, b1, c_above = carry
            g = (BINS_A // 16 - 1) - gi
            w = zero16
            for l in range(16):
                sl = pl.ds(l * 512 + g * 16, 16)
                w = w + histA[sl]
                histA[sl] = zero16
            grp = jnp.sum(w, axis=0)
            rev_w = lax.rev(w, (0,))
            s = plsc.cumsum(rev_w)
            cond = (cum + s) >= kf
            jstar = _scalar(plsc.all_reduce_ffs(cond))
            found_here = jnp.logical_and(jnp.logical_not(done),
                                         (cum + grp) >= kf)
            s_j = _extract(s, jstar)
            w_j = _extract(rev_w, jstar)
            nb1 = jnp.where(found_here, g * 16 + (15 - jstar), b1)
            nca = jnp.where(found_here, cum + s_j - w_j, c_above)
            ncum = jnp.where(done, cum, cum + grp)
            return (ncum, jnp.logical_or(done, found_here), nb1, nca)

        _, _, b1, c_above = sel_a
        k_rem = kf - c_above

        b1v = jnp.zeros((16,), jnp.int32) + b1

        zacc = tuple([zero16] * 10)

        @plsc.parallel_loop(0, NVEC, step=UNROLL, carry=zacc)
        def scan_b(j, carry):
            accs = [list(p) for p in zip(carry[0::2], carry[1::2])]
            for u in range(UNROLL):
                x = row_v[pl.ds(j * 16 + u * 16, 16)]
                key = _key_bits(x)
                binA = plsc.bitcast(key >> jnp.uint32(SH_A), jnp.int32)
                above = binA > b1v
                xx = x * x
                a = accs[u % 5]
                a[0] = a[0] + jnp.where(above, x, zero16)
                a[1] = a[1] + jnp.where(above, xx, zero16)
                eq = binA == b1v
                binB = plsc.bitcast(
                    (key >> jnp.uint32(SH_B)) & jnp.uint32(BINS_B - 1),
                    jnp.int32)
                idx = binB + lane256
                plsc.addupdate_scatter(histB, [idx], ones, mask=eq)
                plsc.addupdate_scatter(histB, [idx + 4096], x, mask=eq)
            return tuple(x for p in accs for x in p)

        S1a_v = scan_b[0] + scan_b[2] + scan_b[4] + scan_b[6] + scan_b[8]
        S2a_v = scan_b[1] + scan_b[3] + scan_b[5] + scan_b[7] + scan_b[9]

        v_in_b1 = binAv == b1v
        v_above = binAv > b1v
        S1a_v = S1a_v - jnp.where(v_above, v16 * (1.0 / 16.0), zero16)
        S2a_v = S2a_v - jnp.where(v_above, v16 * v16 * (1.0 / 16.0), zero16)
        mb = jnp.logical_and(v_in_b1, lane0)
        idxv = binBv + lane256
        plsc.addupdate_scatter(histB, [idxv], -ones, mask=mb)
        plsc.addupdate_scatter(histB, [idxv + 4096], -v16, mask=mb)

        S1a = jnp.sum(S1a_v, axis=0)
        S2a = jnp.sum(S2a_v, axis=0)

        @plsc.parallel_loop(0, BINS_B // 16)
        def red_b(g):
            for f in range(2):
                w = zero16
                for l in range(16):
                    sl = pl.ds(f * 4096 + l * 256 + g * 16, 16)
                    w = w + histB[sl]
                    histB[sl] = zero16
                totB[pl.ds(f * 256 + g * 16, 16)] = w

        init_b = (jnp.float32(0.0), jnp.bool_(False),
                  jnp.float32(0.0), jnp.float32(0.0))

        @plsc.parallel_loop(0, BINS_B // 16, carry=init_b)
        def sel_b(gi, carry):
            cum, done, s1b, s2b = carry
            g = (BINS_B // 16 - 1) - gi
            wc = totB[pl.ds(g * 16, 16)]
            ws = totB[pl.ds(256 + g * 16, 16)]
            mean = ws / jnp.maximum(wc, ones)
            wq = ws * mean
            grp = jnp.sum(wc, axis=0)
            rev_c = lax.rev(wc, (0,))
            rev_s = lax.rev(ws, (0,))
            rev_q = lax.rev(wq, (0,))
            rev_m = lax.rev(mean, (0,))
            sc = plsc.cumsum(rev_c)
            ss = plsc.cumsum(rev_s)
            sq = plsc.cumsum(rev_q)
            cond = (cum + sc) >= k_rem
            jstar = _scalar(plsc.all_reduce_ffs(cond))
            found_here = jnp.logical_and(jnp.logical_not(done),
                                         (cum + grp) >= k_rem)
            c_j = _extract(sc, jstar)
            w_j = _extract(rev_c, jstar)
            s_j = _extract(ss, jstar)
            q_j = _extract(sq, jstar)
            ws_j = _extract(rev_s, jstar)
            wq_j = _extract(rev_q, jstar)
            m_j = _extract(rev_m, jstar)
            r = k_rem - (cum + c_j - w_j)
            add1 = (s_j - ws_j) + r * m_j
            add2 = (q_j - wq_j) + r * m_j * m_j
            ns1 = jnp.where(found_here, s1b + add1,
                            jnp.where(done, s1b, s1b + jnp.sum(ws, axis=0)))
            ns2 = jnp.where(found_here, s2b + add2,
                            jnp.where(done, s2b, s2b + jnp.sum(wq, axis=0)))
            ncum = jnp.where(done, cum, cum + grp)
            return (ncum, jnp.logical_or(done, found_here), ns1, ns2)

        _, _, S1b, S2b = sel_b

        S1 = S1a + S1b
        S2 = S2a + S2b
        neg_term = (S2 + 2.0 * S1 + kf) * jnp.float32(1.0 / K)
        posv = _scalar(v16)
        pos_term = jnp.float32(DELTA) * (posv - 1.0) * (posv - 1.0)
        row_loss = pos_term + neg_term
        return total_acc + jnp.where(lane0, row_loss, 0.0)

    total = lax.fori_loop(0, ROWS_PER_W, row_body, jnp.zeros((16,), jnp.float32))
    acc_v[...] = total
    pltpu.sync_copy(acc_v, out_hbm.at[wid])


def kernel(inputs, targets):
    mesh = plsc.VectorSubcoreMesh(core_axis_name="c", subcore_axis_name="s")
    k = pl.kernel(
        _body,
        out_type=jax.ShapeDtypeStruct((NW, 16), jnp.float32),
        mesh=mesh,
        scratch_types=[
            pltpu.VMEM((N,), jnp.float32),
            pltpu.VMEM((16 * BINS_A,), jnp.float32),
            pltpu.VMEM((2 * 16 * BINS_B,), jnp.float32),
            pltpu.VMEM((2 * BINS_B,), jnp.float32),
            pltpu.VMEM((ROWS_PER_W,), jnp.int32),
            pltpu.VMEM((16,), jnp.float32),
            pltpu.SemaphoreType.DMA,
        ],
        compiler_params=pltpu.CompilerParams(needs_layout_passes=False),
    )
    parts = k(inputs, targets.astype(jnp.int32))
    return jnp.sum(parts) / jnp.float32(M)
